# baseline (speedup 1.0000x reference)
import functools

import jax
import jax.numpy as jnp
from jax import lax
from jax.experimental import pallas as pl
from jax.experimental.pallas import tpu as pltpu
from jax.experimental.pallas import tpu_sc as plsc

BATCH = 16384
DIM = 128
NUM_CORES = 2
NUM_SUBCORES = 16
NW = NUM_CORES * NUM_SUBCORES
B_PER_W = BATCH // NW
CHUNK = 128
NCHUNK = B_PER_W // CHUNK
NSPMEM = 2


def _gather_kernel(idx_hbm, table_hbm, out_hbm, idx_v, spmem, *scr):
    rows = scr[:NCHUNK]
    gsems = scr[NCHUNK:2 * NCHUNK]
    ssems = scr[2 * NCHUNK:3 * NCHUNK]
    wsems = scr[3 * NCHUNK:]
    cid = lax.axis_index("c")
    sid = lax.axis_index("s")
    wid = sid * NUM_CORES + cid
    base = wid * B_PER_W
    pltpu.sync_copy(idx_hbm.at[wid], idx_v)
    gathers = [
        pltpu.async_copy(table_hbm.at[idx_v.at[j]], rows[j], gsems[j])
        for j in range(NCHUNK)
    ]
    stages = [None] * NCHUNK
    writes = [None] * NCHUNK
    for j in range(NCHUNK):
        gathers[j].wait()
        if j < NSPMEM:
            stages[j] = pltpu.async_copy(rows[j], spmem.at[sid, j], ssems[j])
        else:
            writes[j] = pltpu.async_copy(
                rows[j], out_hbm.at[pl.ds(base + j * CHUNK, CHUNK)], wsems[j]
            )
    for j in range(NSPMEM):
        stages[j].wait()
        writes[j] = pltpu.async_copy(
            spmem.at[sid, j], out_hbm.at[pl.ds(base + j * CHUNK, CHUNK)],
            wsems[j],
        )
    for w in writes:
        w.wait()


@jax.jit
def _run(indexes, entity_table):
    mesh = plsc.VectorSubcoreMesh(core_axis_name="c", subcore_axis_name="s")
    scratch = (
        [pltpu.VMEM((NCHUNK, CHUNK), jnp.int32)]
        + [pltpu.VMEM_SHARED((NUM_SUBCORES, NSPMEM, CHUNK, DIM), jnp.float32)]
        + [pltpu.VMEM((CHUNK, DIM), jnp.float32) for _ in range(NCHUNK)]
        + [pltpu.SemaphoreType.DMA for _ in range(3 * NCHUNK)]
    )
    k = functools.partial(
        pl.kernel,
        mesh=mesh,
        out_type=jax.ShapeDtypeStruct((BATCH, DIM), jnp.float32),
        scratch_types=scratch,
    )(_gather_kernel)
    return k(indexes.reshape(NW, NCHUNK, CHUNK), entity_table)


def kernel(indexes, entity_table):
    return _run(indexes.astype(jnp.int32), entity_table)


# speedup vs baseline: 1.0711x; 1.0711x over previous
import functools

import jax
import jax.numpy as jnp
from jax import lax
from jax.experimental import pallas as pl
from jax.experimental.pallas import tpu as pltpu
from jax.experimental.pallas import tpu_sc as plsc

BATCH = 16384
DIM = 128
NUM_CORES = 2
NUM_SUBCORES = 16
NW = NUM_CORES * NUM_SUBCORES
B_PER_W = BATCH // NW


def _gather_kernel(idx_hbm, table_hbm, out_hbm, idx_v, rows_v, sem):
    wid = lax.axis_index("s") * NUM_CORES + lax.axis_index("c")
    base = wid * B_PER_W
    pltpu.sync_copy(idx_hbm.at[pl.ds(base, B_PER_W)], idx_v)
    pltpu.async_copy(table_hbm.at[idx_v], rows_v, sem).wait()
    pltpu.sync_copy(rows_v, out_hbm.at[pl.ds(base, B_PER_W)])


@jax.jit
def _run(indexes, entity_table):
    mesh = plsc.VectorSubcoreMesh(core_axis_name="c", subcore_axis_name="s")
    k = functools.partial(
        pl.kernel,
        mesh=mesh,
        out_type=jax.ShapeDtypeStruct((BATCH, DIM), jnp.float32),
        scratch_types=[
            pltpu.VMEM((B_PER_W,), jnp.int32),
            pltpu.VMEM((B_PER_W, DIM), jnp.float32),
            pltpu.SemaphoreType.DMA,
        ],
    )(_gather_kernel)
    return k(indexes, entity_table)


def kernel(indexes, entity_table):
    return _run(indexes.astype(jnp.int32), entity_table)
